# baseline (speedup 1.0000x reference)
import functools

import jax
import jax.numpy as jnp
from jax import lax
from jax.experimental import pallas as pl
from jax.experimental.pallas import tpu as pltpu
from jax.experimental.pallas import tpu_sc as plsc

N = 10000
E = 320000
D_IN = 128
H = 128
VOCAB = 4096
TD = 64
L = 16
NS = 2
NC = 2

F32 = jnp.float32

NW = 32
NP = 10240
CHUNK = 128
CN = 80
EP = NW * CN * CHUNK

_MESH = plsc.VectorSubcoreMesh(core_axis_name="c", subcore_axis_name="s")


def _wid():
    return lax.axis_index("s") * 2 + lax.axis_index("c")


def _sc_gather(table, idx2, nb=4):
    nrow, ch = idx2.shape
    cn = nrow // NW
    dw = table.shape[1]

    @functools.partial(
        pl.kernel, mesh=_MESH,
        out_type=jax.ShapeDtypeStruct((NW * cn * ch, dw), F32),
        scratch_types=(
            [pltpu.VMEM((cn, ch), jnp.int32)]
            + [pltpu.VMEM((ch, dw), F32) for _ in range(nb)]
            + [pltpu.SemaphoreType.DMA for _ in range(nb)]
        ),
    )
    def k(t_hbm, idx_hbm, out_hbm, gidx, *bufs_sems):
        rows = bufs_sems[:nb]
        sems = bufs_sems[nb:]
        w = _wid()
        pltpu.sync_copy(idx_hbm.at[pl.ds(w * cn, cn)], gidx)

        def start(j, p):
            pltpu.async_copy(t_hbm.at[gidx.at[j]], rows[p], sems[p])

        def drain(j, p):
            pltpu.make_async_copy(t_hbm.at[gidx.at[j]], rows[p], sems[p]).wait()

        def process(j, p):
            drain(j, p)
            pltpu.sync_copy(rows[p], out_hbm.at[pl.ds((w * cn + j) * ch, ch)])

        for i in range(nb - 1):
            start(i, i)

        def body(jj, carry):
            for p in range(nb):
                j = jj * nb + p
                start(lax.rem(j + nb - 1, cn), (p + nb - 1) % nb)
                process(j, p)
            return carry

        lax.fori_loop(0, cn // nb, body, 0)
        for i in range(nb - 1):
            drain(i, i)

    return k(table, idx2)


def _sc_propagate(hp, src2, dst2, zrows, nb=4):
    stripe = NP // 16
    ch = src2.shape[1]
    cn2 = src2.shape[0] // NW
    nstage = 4
    hcn = cn2 // nstage

    @functools.partial(
        pl.kernel, mesh=_MESH,
        out_type=jax.ShapeDtypeStruct((2, NP, H), F32),
        scratch_types=(
            [pltpu.VMEM_SHARED((NP, H), F32),
             pltpu.VMEM((hcn, ch), jnp.int32),
             pltpu.VMEM((hcn, ch), jnp.int32)]
            + [pltpu.VMEM((ch, H), F32) for _ in range(nb)]
            + [pltpu.SemaphoreType.DMA for _ in range(nb)]
        ),
    )
    def k(hp_hbm, src_hbm, dst_hbm, z_hbm, out_hbm, acc, sidx, didx,
          *bufs_sems):
        rows = bufs_sems[:nb]
        sems = bufs_sems[nb:]
        c = lax.axis_index("c")
        s = lax.axis_index("s")
        w = s * 2 + c
        pltpu.sync_copy(z_hbm.at[pl.ds(0, ch)], rows[0])
        for t in range(stripe // ch):
            pltpu.sync_copy(rows[0], acc.at[pl.ds(s * stripe + t * ch, ch)])
        plsc.subcore_barrier()

        def start(j, p):
            pltpu.async_copy(hp_hbm.at[sidx.at[j]], rows[p], sems[p])

        def drain(j, p):
            pltpu.make_async_copy(hp_hbm.at[sidx.at[j]], rows[p], sems[p]).wait()

        def process(j, p):
            drain(j, p)
            pltpu.sync_copy(rows[p], acc.at[didx.at[j]], add=True)

        def body(jj, carry):
            for p in range(nb):
                j = jj * nb + p
                start(lax.rem(j + nb - 1, hcn), (p + nb - 1) % nb)
                process(j, p)
            return carry

        for half in range(nstage):
            base = w * cn2 + half * hcn
            pltpu.sync_copy(src_hbm.at[pl.ds(base, hcn)], sidx)
            pltpu.sync_copy(dst_hbm.at[pl.ds(base, hcn)], didx)
            for i in range(nb - 1):
                start(i, i)
            lax.fori_loop(0, hcn // nb, body, 0)
            for i in range(nb - 1):
                drain(i, i)
        plsc.subcore_barrier()
        for t in range(stripe // ch):
            sl = pl.ds(s * stripe + t * ch, ch)
            pltpu.sync_copy(acc.at[sl], rows[t % 2])
            pltpu.sync_copy(rows[t % 2], out_hbm.at[c, sl])

    return k(hp, src2, dst2, zrows)


def _sc_degree(dst2, orows, zrows):
    stripe = NP // 16

    @functools.partial(
        pl.kernel, mesh=_MESH,
        out_type=jax.ShapeDtypeStruct((2, NP, H), F32),
        scratch_types=[
            pltpu.VMEM_SHARED((NP, H), F32),
            pltpu.VMEM((CN, CHUNK), jnp.int32),
            pltpu.VMEM((CHUNK, H), F32),
            pltpu.VMEM((CHUNK, H), F32),
        ],
    )
    def k(dst_hbm, ones_hbm, z_hbm, out_hbm, acc, didx, ones, zero):
        c = lax.axis_index("c")
        s = lax.axis_index("s")
        w = s * 2 + c
        pltpu.sync_copy(dst_hbm.at[pl.ds(w * CN, CN)], didx)
        pltpu.sync_copy(ones_hbm, ones)
        pltpu.sync_copy(z_hbm, zero)
        for t in range(stripe // CHUNK):
            pltpu.sync_copy(zero, acc.at[pl.ds(s * stripe + t * CHUNK, CHUNK)])
        plsc.subcore_barrier()

        def body(j, carry):
            pltpu.sync_copy(ones, acc.at[didx.at[j]], add=True)
            return carry

        lax.fori_loop(0, CN, body, 0)
        plsc.subcore_barrier()
        for t in range(stripe // CHUNK):
            sl = pl.ds(s * stripe + t * CHUNK, CHUNK)
            pltpu.sync_copy(acc.at[sl], ones)
            pltpu.sync_copy(ones, out_hbm.at[c, sl])

    return k(dst2, orows, zrows)


def _deg_reduce_body(p0_ref, p1_ref, o_ref):
    s = p0_ref[0][:, 0:1] + p1_ref[0][:, 0:1]
    o_ref[...] = lax.rsqrt(s + 1.0)


def _dinv_from_parts(parts, *, bn=2048):
    return pl.pallas_call(
        _deg_reduce_body,
        grid=(NP // bn,),
        in_specs=[
            pl.BlockSpec((1, bn, H), lambda i: (0, i, 0)),
            pl.BlockSpec((1, bn, H), lambda i: (1, i, 0)),
        ],
        out_specs=pl.BlockSpec((bn, 1), lambda i: (i, 0)),
        out_shape=jax.ShapeDtypeStruct((NP, 1), F32),
    )(parts, parts)


def _mm_body(a_ref, w_ref, b_ref, o_ref, *, relu, scale_ref=None):
    acc = jnp.dot(a_ref[...], w_ref[...], preferred_element_type=F32)
    acc = acc + b_ref[...]
    if scale_ref is not None:
        acc = acc * scale_ref[...]
    if relu:
        acc = jnp.maximum(acc, 0.0)
    o_ref[...] = acc


def _mm(a, w, b, *, relu=False, scale=None, bm=2000):
    m, k = a.shape
    n = w.shape[1]
    assert m % bm == 0, (m, bm)
    b2 = jnp.reshape(b, (1, n))
    in_specs = [
        pl.BlockSpec((bm, k), lambda i: (i, 0)),
        pl.BlockSpec((k, n), lambda i: (0, 0)),
        pl.BlockSpec((1, n), lambda i: (0, 0)),
    ]
    args = [a, w, b2]
    if scale is not None:
        in_specs.append(pl.BlockSpec((bm, 1), lambda i: (i, 0)))
        args.append(jnp.reshape(scale, (m, 1)))
        body = functools.partial(_mm_body_scaled, relu=relu)
    else:
        body = functools.partial(_mm_body, relu=relu)
    return pl.pallas_call(
        body,
        grid=(m // bm,),
        in_specs=in_specs,
        out_specs=pl.BlockSpec((bm, n), lambda i: (i, 0)),
        out_shape=jax.ShapeDtypeStruct((m, n), F32),
    )(*args)


def _mm_body_scaled(a_ref, w_ref, b_ref, s_ref, o_ref, *, relu):
    acc = jnp.dot(a_ref[...], w_ref[...], preferred_element_type=F32)
    acc = acc + b_ref[...]
    acc = acc * s_ref[...]
    if relu:
        acc = jnp.maximum(acc, 0.0)
    o_ref[...] = acc


def _gcn_combine_body(p0_ref, p1_ref, hp_ref, d_ref, b_ref, o_ref):
    p = p0_ref[0] + p1_ref[0]
    o_ref[...] = jnp.maximum(
        (p + hp_ref[...]) * d_ref[...] + b_ref[...], 0.0)


def _gcn_combine(parts, hp, dinv, b, *, bm=2000):
    m, n = hp.shape
    return pl.pallas_call(
        _gcn_combine_body,
        grid=(m // bm,),
        in_specs=[
            pl.BlockSpec((1, bm, n), lambda i: (0, i, 0)),
            pl.BlockSpec((1, bm, n), lambda i: (1, i, 0)),
            pl.BlockSpec((bm, n), lambda i: (i, 0)),
            pl.BlockSpec((bm, 1), lambda i: (i, 0)),
            pl.BlockSpec((1, n), lambda i: (0, 0)),
        ],
        out_specs=pl.BlockSpec((bm, n), lambda i: (i, 0)),
        out_shape=jax.ShapeDtypeStruct((m, n), F32),
    )(parts, parts, hp, jnp.reshape(dinv, (m, 1)), jnp.reshape(b, (1, n)))


def _gru_body(xe_ref, wih_ref, whh_ref, bih_ref, bhh_ref, o_ref):
    bn = xe_ref.shape[0]
    wih = wih_ref[...]
    whh = whh_ref[...]
    bih = bih_ref[...]
    bhh = bhh_ref[...]
    xe = xe_ref[...]

    h = jnp.zeros((bn, H), F32)
    acc = jnp.zeros((bn, H), F32)
    for t in range(L):
        xt = xe[:, t, :]
        gi = jnp.dot(xt, wih, preferred_element_type=F32) + bih
        gh = jnp.dot(h, whh, preferred_element_type=F32) + bhh
        i_r = gi[:, 0:H]
        i_z = gi[:, H:2 * H]
        i_n = gi[:, 2 * H:3 * H]
        h_r = gh[:, 0:H]
        h_z = gh[:, H:2 * H]
        h_n = gh[:, 2 * H:3 * H]
        r = jax.nn.sigmoid(i_r + h_r)
        z = jax.nn.sigmoid(i_z + h_z)
        ncand = jnp.tanh(i_n + r * h_n)
        h = (1.0 - z) * ncand + z * h
        acc = acc + h
    o_ref[...] = acc


def _gru_sum(xe, gWih, gWhh, gbih, gbhh, *, bn=1000):
    n = xe.shape[0]
    wih = jnp.pad(gWih.T, ((0, 128 - TD), (0, 0)))
    whh = gWhh.T
    return pl.pallas_call(
        _gru_body,
        grid=(n // bn,),
        in_specs=[
            pl.BlockSpec((bn, L, 128), lambda i: (i, 0, 0)),
            pl.BlockSpec((128, 3 * H), lambda i: (0, 0)),
            pl.BlockSpec((H, 3 * H), lambda i: (0, 0)),
            pl.BlockSpec((1, 3 * H), lambda i: (0, 0)),
            pl.BlockSpec((1, 3 * H), lambda i: (0, 0)),
        ],
        out_specs=pl.BlockSpec((bn, H), lambda i: (i, 0)),
        out_shape=jax.ShapeDtypeStruct((n, H), F32),
    )(xe, wih, whh, jnp.reshape(gbih, (1, 3 * H)), jnp.reshape(gbhh, (1, 3 * H)))


def _im2col_s2(x):
    hi, wi, c = x.shape
    ho, wo = hi // 2, wi // 2
    xp = jnp.pad(x, ((0, 1), (0, 1), (0, 0)))
    slabs = []
    for dy in range(3):
        for dx in range(3):
            slabs.append(xp[dy:dy + hi:2, dx:dx + wi:2, :])
    a = jnp.stack(slabs, axis=2)
    return jnp.reshape(a, (ho * wo, 9 * c))


def _conv_gemm(x, k, cb, *, bm):
    hi, wi, cin = x.shape
    cout = k.shape[0]
    a = _im2col_s2(x)
    wmat = jnp.reshape(jnp.transpose(k, (2, 3, 1, 0)), (9 * cin, cout))
    out = _mm(a, wmat, cb, relu=True, bm=bm)
    return jnp.reshape(out, (hi // 2, wi // 2, cout))


def _edge_body(us_ref, ud_ref, bcat_ref, r1_ref, r1b_ref, r2_ref, r2b_ref,
               o_ref):
    e = jnp.maximum(us_ref[...] + ud_ref[...] + bcat_ref[...], 0.0)
    h = jnp.maximum(
        jnp.dot(e, r1_ref[...], preferred_element_type=F32) + r1b_ref[...], 0.0)
    logits = jnp.dot(h, r2_ref[...], preferred_element_type=F32) + r2b_ref[...]
    m = jnp.max(logits, axis=1, keepdims=True)
    lse = jnp.log(jnp.sum(jnp.exp(logits - m), axis=1, keepdims=True)) + m
    o_ref[...] = logits - lse


def _edge_head(us, ud, bcat, r1W, r1b, r2W, r2b, *, be=3200):
    e = us.shape[0]
    return pl.pallas_call(
        _edge_body,
        grid=(e // be,),
        in_specs=[
            pl.BlockSpec((be, 3 * H), lambda i: (i, 0)),
            pl.BlockSpec((be, 3 * H), lambda i: (i, 0)),
            pl.BlockSpec((1, 3 * H), lambda i: (0, 0)),
            pl.BlockSpec((3 * H, H), lambda i: (0, 0)),
            pl.BlockSpec((1, H), lambda i: (0, 0)),
            pl.BlockSpec((H, NC), lambda i: (0, 0)),
            pl.BlockSpec((1, NC), lambda i: (0, 0)),
        ],
        out_specs=pl.BlockSpec((be, NC), lambda i: (i, 0)),
        out_shape=jax.ShapeDtypeStruct((e, NC), F32),
    )(us, ud, jnp.reshape(bcat, (1, 3 * H)), r1W, jnp.reshape(r1b, (1, H)),
      r2W, jnp.reshape(r2b, (1, NC)))


def _gcn_layer(x, src2, dst2, dinv, zrows, W, b):
    hp = _mm(x, W, jnp.zeros((W.shape[1],), F32), scale=dinv)
    parts = _sc_propagate(hp, src2, dst2, zrows)
    return _gcn_combine(parts, hp, dinv, b)


def kernel(x, edge_index, xtext, img, nodenum, pos, cell_wh, W1, b1, W2, b2,
           Wt1, bt1, Wt2, bt2, emb, gWih, gWhh, gbih, gbhh, K1, cb1, K2, cb2,
           K3, cb3, lpW, lpb, ltW, ltb, liW, lib, r1W, r1b, r2W, r2b):
    src = edge_index[0]
    dst = edge_index[1]
    pad_e = EP - E
    i32 = jnp.int32
    srcp = jnp.concatenate([src, jnp.zeros((pad_e,), i32)])
    dstp_sc = jnp.concatenate([dst, jnp.full((pad_e,), NP - 1, i32)])
    dstp_u = jnp.concatenate([dst, jnp.zeros((pad_e,), i32)])
    src64 = jnp.reshape(srcp, (NW * CN * 2, CHUNK // 2))
    dst64sc = jnp.reshape(dstp_sc, (NW * CN * 2, CHUNK // 2))
    dst64u = jnp.reshape(dstp_u, (NW * CN * 2, CHUNK // 2))
    dst2sc = jnp.reshape(dstp_sc, (NW * CN, CHUNK))
    zrows = jnp.zeros((CHUNK, H), F32)

    orows = jnp.ones((CHUNK, H), F32)
    degp = _sc_degree(dst2sc, orows, zrows)
    dinv = _dinv_from_parts(degp)[:N]

    pf = _gcn_layer(x, src64, dst64sc, dinv, zrows, W1, b1)
    pf = _gcn_layer(pf, src64, dst64sc, dinv, zrows, W2, b2)

    nlp = NW * 40 * CHUNK
    tok = jnp.reshape(xtext, (-1,))
    tok2 = jnp.reshape(
        jnp.concatenate([tok, jnp.zeros((nlp - N * L,), i32)]), (NW * 40, CHUNK))
    emb128 = jnp.pad(emb, ((0, 0), (0, 128 - TD)))
    xef = _sc_gather(emb128, tok2)
    xe = jnp.reshape(xef[:N * L], (N, L, 128))
    tf = _gru_sum(xe, gWih, gWhh, gbih, gbhh)
    tf = _gcn_layer(tf, src64, dst64sc, dinv, zrows, Wt1, bt1)
    tf = _gcn_layer(tf, src64, dst64sc, dinv, zrows, Wt2, bt2)

    xim = jnp.transpose(img[0], (1, 2, 0))
    fm = _conv_gemm(xim, K1, cb1, bm=4096)
    fm = _conv_gemm(fm, K2, cb2, bm=4096)
    fm = _conv_gemm(fm, K3, cb3, bm=4096)
    fmflat = jnp.reshape(fm, (64 * 64, H))

    hf = wf = 64
    idx_parts = []
    for s in range(NS):
        t = (s + 0.5) / NS
        q = pos - cell_wh / 2.0 + t * cell_wh
        ix = jnp.clip((jnp.clip(q[:, 0], 0.0, 1.0) * wf).astype(i32), 0, wf - 1)
        iy = jnp.clip((jnp.clip(q[:, 1], 0.0, 1.0) * hf).astype(i32), 0, hf - 1)
        idx_parts.append(
            jnp.concatenate([iy * wf + ix, jnp.zeros((NP - N,), i32)]))
    idx_parts.append(jnp.zeros((NW * 8 * CHUNK - 2 * NP,), i32))
    idx2 = jnp.reshape(jnp.concatenate(idx_parts), (NW * 8, CHUNK))
    g = _sc_gather(fmflat, idx2)
    g0 = g[:N]
    g1 = g[NP:NP + N]

    feat = jnp.concatenate([pf, tf, g0, g1], axis=1)
    z = jnp.zeros((H, H), F32)
    ws = jnp.concatenate([
        jnp.concatenate([lpW[:H], z, z], axis=1),
        jnp.concatenate([z, ltW[:H], z], axis=1),
        jnp.concatenate([z, z, liW[0:H]], axis=1),
        jnp.concatenate([z, z, liW[H:2 * H]], axis=1),
    ], axis=0)
    wd = jnp.concatenate([
        jnp.concatenate([lpW[H:], z, z], axis=1),
        jnp.concatenate([z, ltW[H:], z], axis=1),
        jnp.concatenate([z, z, liW[2 * H:3 * H]], axis=1),
        jnp.concatenate([z, z, liW[3 * H:4 * H]], axis=1),
    ], axis=0)
    zb = jnp.zeros((3 * H,), F32)
    S = _mm(feat, ws, zb)
    D = _mm(feat, wd, zb)

    us = _sc_gather(S, src64)
    ud = _sc_gather(D, dst64u)
    bcat = jnp.concatenate([lpb, ltb, lib])
    out = _edge_head(us, ud, bcat, r1W, r1b, r2W, r2b, be=4096)
    return out[:E]


# speedup vs baseline: 1.0376x; 1.0376x over previous
import functools

import jax
import jax.numpy as jnp
from jax import lax
from jax.experimental import pallas as pl
from jax.experimental.pallas import tpu as pltpu
from jax.experimental.pallas import tpu_sc as plsc

N = 10000
E = 320000
D_IN = 128
H = 128
VOCAB = 4096
TD = 64
L = 16
NS = 2
NC = 2

F32 = jnp.float32

NW = 32
NP = 10240
CHUNK = 128
CN = 80
EP = NW * CN * CHUNK

_MESH = plsc.VectorSubcoreMesh(core_axis_name="c", subcore_axis_name="s")


def _wid():
    return lax.axis_index("s") * 2 + lax.axis_index("c")


def _sc_gather(table, idx2, nb=4):
    nrow, ch = idx2.shape
    cn = nrow // NW
    dw = table.shape[1]

    @functools.partial(
        pl.kernel, mesh=_MESH,
        out_type=jax.ShapeDtypeStruct((NW * cn * ch, dw), F32),
        scratch_types=(
            [pltpu.VMEM((cn, ch), jnp.int32)]
            + [pltpu.VMEM((ch, dw), F32) for _ in range(nb)]
            + [pltpu.SemaphoreType.DMA for _ in range(nb)]
        ),
    )
    def k(t_hbm, idx_hbm, out_hbm, gidx, *bufs_sems):
        rows = bufs_sems[:nb]
        sems = bufs_sems[nb:]
        w = _wid()
        pltpu.sync_copy(idx_hbm.at[pl.ds(w * cn, cn)], gidx)

        def start(j, p):
            pltpu.async_copy(t_hbm.at[gidx.at[j]], rows[p], sems[p])

        def drain(j, p):
            pltpu.make_async_copy(t_hbm.at[gidx.at[j]], rows[p], sems[p]).wait()

        def process(j, p):
            drain(j, p)
            pltpu.sync_copy(rows[p], out_hbm.at[pl.ds((w * cn + j) * ch, ch)])

        for i in range(nb - 1):
            start(i, i)

        def body(jj, carry):
            for p in range(nb):
                j = jj * nb + p
                start(lax.rem(j + nb - 1, cn), (p + nb - 1) % nb)
                process(j, p)
            return carry

        lax.fori_loop(0, cn // nb, body, 0)
        for i in range(nb - 1):
            drain(i, i)

    return k(table, idx2)


def _sc_propagate(hp, src2, dst2, zrows, nb=2):
    stripe = NP // 16
    ch = src2.shape[1]
    cn2 = src2.shape[0] // NW
    nstage = 2
    hcn = cn2 // nstage

    @functools.partial(
        pl.kernel, mesh=_MESH,
        out_type=jax.ShapeDtypeStruct((2, NP, H), F32),
        scratch_types=(
            [pltpu.VMEM_SHARED((NP, H), F32),
             pltpu.VMEM((hcn, ch), jnp.int32),
             pltpu.VMEM((hcn, ch), jnp.int32)]
            + [pltpu.VMEM((ch, H), F32) for _ in range(nb)]
            + [pltpu.SemaphoreType.DMA for _ in range(nb)]
        ),
    )
    def k(hp_hbm, src_hbm, dst_hbm, z_hbm, out_hbm, acc, sidx, didx,
          *bufs_sems):
        rows = bufs_sems[:nb]
        sems = bufs_sems[nb:]
        c = lax.axis_index("c")
        s = lax.axis_index("s")
        w = s * 2 + c
        pltpu.sync_copy(z_hbm.at[pl.ds(0, ch)], rows[0])
        for t in range(stripe // ch):
            pltpu.sync_copy(rows[0], acc.at[pl.ds(s * stripe + t * ch, ch)])
        plsc.subcore_barrier()

        def start(j, p):
            pltpu.async_copy(hp_hbm.at[sidx.at[j]], rows[p], sems[p])

        def drain(j, p):
            pltpu.make_async_copy(hp_hbm.at[sidx.at[j]], rows[p], sems[p]).wait()

        def process(j, p):
            drain(j, p)
            pltpu.sync_copy(rows[p], acc.at[didx.at[j]], add=True)

        def body(jj, carry):
            for p in range(nb):
                j = jj * nb + p
                start(lax.rem(j + nb - 1, hcn), (p + nb - 1) % nb)
                process(j, p)
            return carry

        for half in range(nstage):
            base = w * cn2 + half * hcn
            pltpu.sync_copy(src_hbm.at[pl.ds(base, hcn)], sidx)
            pltpu.sync_copy(dst_hbm.at[pl.ds(base, hcn)], didx)
            for i in range(nb - 1):
                start(i, i)
            lax.fori_loop(0, hcn // nb, body, 0)
            for i in range(nb - 1):
                drain(i, i)
        plsc.subcore_barrier()
        for t in range(stripe // ch):
            sl = pl.ds(s * stripe + t * ch, ch)
            pltpu.sync_copy(acc.at[sl], rows[t % 2])
            pltpu.sync_copy(rows[t % 2], out_hbm.at[c, sl])

    return k(hp, src2, dst2, zrows)


def _sc_degree(dst2, orows, zrows):
    stripe = NP // 16

    @functools.partial(
        pl.kernel, mesh=_MESH,
        out_type=jax.ShapeDtypeStruct((2, NP, H), F32),
        scratch_types=[
            pltpu.VMEM_SHARED((NP, H), F32),
            pltpu.VMEM((CN, CHUNK), jnp.int32),
            pltpu.VMEM((CHUNK, H), F32),
            pltpu.VMEM((CHUNK, H), F32),
            pltpu.SemaphoreType.DMA,
        ],
    )
    def k(dst_hbm, ones_hbm, z_hbm, out_hbm, acc, didx, ones, zero, sem):
        c = lax.axis_index("c")
        s = lax.axis_index("s")
        w = s * 2 + c
        pltpu.sync_copy(dst_hbm.at[pl.ds(w * CN, CN)], didx)
        pltpu.sync_copy(ones_hbm, ones)
        pltpu.sync_copy(z_hbm, zero)
        for t in range(stripe // CHUNK):
            pltpu.sync_copy(zero, acc.at[pl.ds(s * stripe + t * CHUNK, CHUNK)])
        plsc.subcore_barrier()

        kf = 8

        def body(jj, carry):
            for p in range(kf):
                pltpu.async_copy(
                    ones, acc.at[didx.at[jj * kf + p]], sem, add=True)
            for p in range(kf):
                pltpu.make_async_copy(
                    ones, acc.at[didx.at[jj * kf + p]], sem).wait()
            return carry

        lax.fori_loop(0, CN // kf, body, 0)
        plsc.subcore_barrier()
        for t in range(stripe // CHUNK):
            sl = pl.ds(s * stripe + t * CHUNK, CHUNK)
            pltpu.sync_copy(acc.at[sl], ones)
            pltpu.sync_copy(ones, out_hbm.at[c, sl])

    return k(dst2, orows, zrows)


def _deg_reduce_body(p0_ref, p1_ref, o_ref):
    s = p0_ref[0][:, 0:1] + p1_ref[0][:, 0:1]
    o_ref[...] = lax.rsqrt(s + 1.0)


def _dinv_from_parts(parts, *, bn=2048):
    return pl.pallas_call(
        _deg_reduce_body,
        grid=(NP // bn,),
        in_specs=[
            pl.BlockSpec((1, bn, H), lambda i: (0, i, 0)),
            pl.BlockSpec((1, bn, H), lambda i: (1, i, 0)),
        ],
        out_specs=pl.BlockSpec((bn, 1), lambda i: (i, 0)),
        out_shape=jax.ShapeDtypeStruct((NP, 1), F32),
    )(parts, parts)


def _mm_body(a_ref, w_ref, b_ref, o_ref, *, relu, scale_ref=None):
    acc = jnp.dot(a_ref[...], w_ref[...], preferred_element_type=F32)
    acc = acc + b_ref[...]
    if scale_ref is not None:
        acc = acc * scale_ref[...]
    if relu:
        acc = jnp.maximum(acc, 0.0)
    o_ref[...] = acc


def _mm(a, w, b, *, relu=False, scale=None, bm=2000):
    m, k = a.shape
    n = w.shape[1]
    assert m % bm == 0, (m, bm)
    b2 = jnp.reshape(b, (1, n))
    in_specs = [
        pl.BlockSpec((bm, k), lambda i: (i, 0)),
        pl.BlockSpec((k, n), lambda i: (0, 0)),
        pl.BlockSpec((1, n), lambda i: (0, 0)),
    ]
    args = [a, w, b2]
    if scale is not None:
        in_specs.append(pl.BlockSpec((bm, 1), lambda i: (i, 0)))
        args.append(jnp.reshape(scale, (m, 1)))
        body = functools.partial(_mm_body_scaled, relu=relu)
    else:
        body = functools.partial(_mm_body, relu=relu)
    return pl.pallas_call(
        body,
        grid=(m // bm,),
        in_specs=in_specs,
        out_specs=pl.BlockSpec((bm, n), lambda i: (i, 0)),
        out_shape=jax.ShapeDtypeStruct((m, n), F32),
    )(*args)


def _mm_body_scaled(a_ref, w_ref, b_ref, s_ref, o_ref, *, relu):
    acc = jnp.dot(a_ref[...], w_ref[...], preferred_element_type=F32)
    acc = acc + b_ref[...]
    acc = acc * s_ref[...]
    if relu:
        acc = jnp.maximum(acc, 0.0)
    o_ref[...] = acc


def _gcn_combine_body(p0_ref, p1_ref, hp_ref, d_ref, b_ref, o_ref):
    p = p0_ref[0] + p1_ref[0]
    o_ref[...] = jnp.maximum(
        (p + hp_ref[...]) * d_ref[...] + b_ref[...], 0.0)


def _gcn_combine(parts, hp, dinv, b, *, bm=2000):
    m, n = hp.shape
    return pl.pallas_call(
        _gcn_combine_body,
        grid=(m // bm,),
        in_specs=[
            pl.BlockSpec((1, bm, n), lambda i: (0, i, 0)),
            pl.BlockSpec((1, bm, n), lambda i: (1, i, 0)),
            pl.BlockSpec((bm, n), lambda i: (i, 0)),
            pl.BlockSpec((bm, 1), lambda i: (i, 0)),
            pl.BlockSpec((1, n), lambda i: (0, 0)),
        ],
        out_specs=pl.BlockSpec((bm, n), lambda i: (i, 0)),
        out_shape=jax.ShapeDtypeStruct((m, n), F32),
    )(parts, parts, hp, jnp.reshape(dinv, (m, 1)), jnp.reshape(b, (1, n)))


def _gru_body(xe_ref, wih_ref, whh_ref, bih_ref, bhh_ref, o_ref):
    bn = xe_ref.shape[0]
    wih = wih_ref[...]
    whh = whh_ref[...]
    bih = bih_ref[...]
    bhh = bhh_ref[...]
    xe = xe_ref[...]

    h = jnp.zeros((bn, H), F32)
    acc = jnp.zeros((bn, H), F32)
    for t in range(L):
        xt = xe[:, t, :]
        gi = jnp.dot(xt, wih, preferred_element_type=F32) + bih
        gh = jnp.dot(h, whh, preferred_element_type=F32) + bhh
        i_r = gi[:, 0:H]
        i_z = gi[:, H:2 * H]
        i_n = gi[:, 2 * H:3 * H]
        h_r = gh[:, 0:H]
        h_z = gh[:, H:2 * H]
        h_n = gh[:, 2 * H:3 * H]
        r = jax.nn.sigmoid(i_r + h_r)
        z = jax.nn.sigmoid(i_z + h_z)
        ncand = jnp.tanh(i_n + r * h_n)
        h = (1.0 - z) * ncand + z * h
        acc = acc + h
    o_ref[...] = acc


def _gru_sum(xe, gWih, gWhh, gbih, gbhh, *, bn=1000):
    n = xe.shape[0]
    wih = jnp.pad(gWih.T, ((0, 128 - TD), (0, 0)))
    whh = gWhh.T
    return pl.pallas_call(
        _gru_body,
        grid=(n // bn,),
        in_specs=[
            pl.BlockSpec((bn, L, 128), lambda i: (i, 0, 0)),
            pl.BlockSpec((128, 3 * H), lambda i: (0, 0)),
            pl.BlockSpec((H, 3 * H), lambda i: (0, 0)),
            pl.BlockSpec((1, 3 * H), lambda i: (0, 0)),
            pl.BlockSpec((1, 3 * H), lambda i: (0, 0)),
        ],
        out_specs=pl.BlockSpec((bn, H), lambda i: (i, 0)),
        out_shape=jax.ShapeDtypeStruct((n, H), F32),
    )(xe, wih, whh, jnp.reshape(gbih, (1, 3 * H)), jnp.reshape(gbhh, (1, 3 * H)))


def _im2col_s2(x):
    hi, wi, c = x.shape
    ho, wo = hi // 2, wi // 2
    xp = jnp.pad(x, ((0, 1), (0, 1), (0, 0)))
    slabs = []
    for dy in range(3):
        for dx in range(3):
            slabs.append(xp[dy:dy + hi:2, dx:dx + wi:2, :])
    a = jnp.stack(slabs, axis=2)
    return jnp.reshape(a, (ho * wo, 9 * c))


def _conv_gemm(x, k, cb, *, bm):
    hi, wi, cin = x.shape
    cout = k.shape[0]
    a = _im2col_s2(x)
    wmat = jnp.reshape(jnp.transpose(k, (2, 3, 1, 0)), (9 * cin, cout))
    out = _mm(a, wmat, cb, relu=True, bm=bm)
    return jnp.reshape(out, (hi // 2, wi // 2, cout))


def _edge_body(us_ref, ud_ref, bcat_ref, r1_ref, r1b_ref, r2_ref, r2b_ref,
               o_ref):
    e = jnp.maximum(us_ref[...] + ud_ref[...] + bcat_ref[...], 0.0)
    h = jnp.maximum(
        jnp.dot(e, r1_ref[...], preferred_element_type=F32) + r1b_ref[...], 0.0)
    logits = jnp.dot(h, r2_ref[...], preferred_element_type=F32) + r2b_ref[...]
    m = jnp.max(logits, axis=1, keepdims=True)
    lse = jnp.log(jnp.sum(jnp.exp(logits - m), axis=1, keepdims=True)) + m
    o_ref[...] = logits - lse


def _edge_head(us, ud, bcat, r1W, r1b, r2W, r2b, *, be=3200):
    e = us.shape[0]
    return pl.pallas_call(
        _edge_body,
        grid=(e // be,),
        in_specs=[
            pl.BlockSpec((be, 3 * H), lambda i: (i, 0)),
            pl.BlockSpec((be, 3 * H), lambda i: (i, 0)),
            pl.BlockSpec((1, 3 * H), lambda i: (0, 0)),
            pl.BlockSpec((3 * H, H), lambda i: (0, 0)),
            pl.BlockSpec((1, H), lambda i: (0, 0)),
            pl.BlockSpec((H, NC), lambda i: (0, 0)),
            pl.BlockSpec((1, NC), lambda i: (0, 0)),
        ],
        out_specs=pl.BlockSpec((be, NC), lambda i: (i, 0)),
        out_shape=jax.ShapeDtypeStruct((e, NC), F32),
    )(us, ud, jnp.reshape(bcat, (1, 3 * H)), r1W, jnp.reshape(r1b, (1, H)),
      r2W, jnp.reshape(r2b, (1, NC)))


def _gcn_layer(x, src2, dst2, dinv, zrows, W, b):
    hp = _mm(x, W, jnp.zeros((W.shape[1],), F32), scale=dinv)
    parts = _sc_propagate(hp, src2, dst2, zrows)
    return _gcn_combine(parts, hp, dinv, b)


def kernel(x, edge_index, xtext, img, nodenum, pos, cell_wh, W1, b1, W2, b2,
           Wt1, bt1, Wt2, bt2, emb, gWih, gWhh, gbih, gbhh, K1, cb1, K2, cb2,
           K3, cb3, lpW, lpb, ltW, ltb, liW, lib, r1W, r1b, r2W, r2b):
    src = edge_index[0]
    dst = edge_index[1]
    pad_e = EP - E
    i32 = jnp.int32
    srcp = jnp.concatenate([src, jnp.zeros((pad_e,), i32)])
    dstp_sc = jnp.concatenate([dst, jnp.full((pad_e,), NP - 1, i32)])
    dstp_u = jnp.concatenate([dst, jnp.zeros((pad_e,), i32)])
    src2 = jnp.reshape(srcp, (NW * CN, CHUNK))
    dst2sc = jnp.reshape(dstp_sc, (NW * CN, CHUNK))
    dst2u = jnp.reshape(dstp_u, (NW * CN, CHUNK))
    zrows = jnp.zeros((CHUNK, H), F32)

    orows = jnp.ones((CHUNK, H), F32)
    degp = _sc_degree(dst2sc, orows, zrows)
    dinv = _dinv_from_parts(degp)[:N]

    pf = _gcn_layer(x, src2, dst2sc, dinv, zrows, W1, b1)
    pf = _gcn_layer(pf, src2, dst2sc, dinv, zrows, W2, b2)

    nlp = NW * 40 * CHUNK
    tok = jnp.reshape(xtext, (-1,))
    tok2 = jnp.reshape(
        jnp.concatenate([tok, jnp.zeros((nlp - N * L,), i32)]), (NW * 40, CHUNK))
    emb128 = jnp.pad(emb, ((0, 0), (0, 128 - TD)))
    xef = _sc_gather(emb128, tok2)
    xe = jnp.reshape(xef[:N * L], (N, L, 128))
    tf = _gru_sum(xe, gWih, gWhh, gbih, gbhh)
    tf = _gcn_layer(tf, src2, dst2sc, dinv, zrows, Wt1, bt1)
    tf = _gcn_layer(tf, src2, dst2sc, dinv, zrows, Wt2, bt2)

    xim = jnp.transpose(img[0], (1, 2, 0))
    fm = _conv_gemm(xim, K1, cb1, bm=4096)
    fm = _conv_gemm(fm, K2, cb2, bm=4096)
    fm = _conv_gemm(fm, K3, cb3, bm=4096)
    fmflat = jnp.reshape(fm, (64 * 64, H))

    hf = wf = 64
    idx_parts = []
    for s in range(NS):
        t = (s + 0.5) / NS
        q = pos - cell_wh / 2.0 + t * cell_wh
        ix = jnp.clip((jnp.clip(q[:, 0], 0.0, 1.0) * wf).astype(i32), 0, wf - 1)
        iy = jnp.clip((jnp.clip(q[:, 1], 0.0, 1.0) * hf).astype(i32), 0, hf - 1)
        idx_parts.append(
            jnp.concatenate([iy * wf + ix, jnp.zeros((NP - N,), i32)]))
    idx_parts.append(jnp.zeros((NW * 8 * CHUNK - 2 * NP,), i32))
    idx2 = jnp.reshape(jnp.concatenate(idx_parts), (NW * 8, CHUNK))
    g = _sc_gather(fmflat, idx2)
    g0 = g[:N]
    g1 = g[NP:NP + N]

    feat = jnp.concatenate([pf, tf, g0, g1], axis=1)
    z = jnp.zeros((H, H), F32)
    ws = jnp.concatenate([
        jnp.concatenate([lpW[:H], z, z], axis=1),
        jnp.concatenate([z, ltW[:H], z], axis=1),
        jnp.concatenate([z, z, liW[0:H]], axis=1),
        jnp.concatenate([z, z, liW[H:2 * H]], axis=1),
    ], axis=0)
    wd = jnp.concatenate([
        jnp.concatenate([lpW[H:], z, z], axis=1),
        jnp.concatenate([z, ltW[H:], z], axis=1),
        jnp.concatenate([z, z, liW[2 * H:3 * H]], axis=1),
        jnp.concatenate([z, z, liW[3 * H:4 * H]], axis=1),
    ], axis=0)
    zb = jnp.zeros((3 * H,), F32)
    S = _mm(feat, ws, zb)
    D = _mm(feat, wd, zb)

    us = _sc_gather(S, src2, nb=2)
    ud = _sc_gather(D, dst2u, nb=2)
    bcat = jnp.concatenate([lpb, ltb, lib])
    out = _edge_head(us, ud, bcat, r1W, r1b, r2W, r2b, be=4096)
    return out[:E]


# speedup vs baseline: 1.0586x; 1.0202x over previous
import functools

import jax
import jax.numpy as jnp
from jax import lax
from jax.experimental import pallas as pl
from jax.experimental.pallas import tpu as pltpu
from jax.experimental.pallas import tpu_sc as plsc

N = 10000
E = 320000
D_IN = 128
H = 128
VOCAB = 4096
TD = 64
L = 16
NS = 2
NC = 2

F32 = jnp.float32

NW = 32
NP = 10240
CHUNK = 128
CN = 80
EP = NW * CN * CHUNK

_MESH = plsc.VectorSubcoreMesh(core_axis_name="c", subcore_axis_name="s")


def _wid():
    return lax.axis_index("s") * 2 + lax.axis_index("c")


def _sc_gather(table, idx2, nb=2):
    nrow, ch = idx2.shape
    cn = nrow // NW
    dw = table.shape[1]

    @functools.partial(
        pl.kernel, mesh=_MESH,
        out_type=jax.ShapeDtypeStruct((NW * cn * ch, dw), F32),
        scratch_types=(
            [pltpu.VMEM((cn, ch), jnp.int32)]
            + [pltpu.VMEM((ch, dw), F32) for _ in range(nb)]
            + [pltpu.SemaphoreType.DMA for _ in range(nb)]
        ),
    )
    def k(t_hbm, idx_hbm, out_hbm, gidx, *bufs_sems):
        rows = bufs_sems[:nb]
        sems = bufs_sems[nb:]
        w = _wid()
        pltpu.sync_copy(idx_hbm.at[pl.ds(w * cn, cn)], gidx)

        def start(j, p):
            pltpu.async_copy(t_hbm.at[gidx.at[j]], rows[p], sems[p])

        def drain(j, p):
            pltpu.make_async_copy(t_hbm.at[gidx.at[j]], rows[p], sems[p]).wait()

        def process(j, p):
            drain(j, p)
            pltpu.sync_copy(rows[p], out_hbm.at[pl.ds((w * cn + j) * ch, ch)])

        for i in range(nb - 1):
            start(i, i)

        def body(jj, carry):
            for p in range(nb):
                j = jj * nb + p
                start(lax.rem(j + nb - 1, cn), (p + nb - 1) % nb)
                process(j, p)
            return carry

        lax.fori_loop(0, cn // nb, body, 0)
        for i in range(nb - 1):
            drain(i, i)

    return k(table, idx2)


def _sc_propagate(hp, src2, dst2, zrows, nb=2):
    stripe = NP // 16
    ch = src2.shape[1]
    cn2 = src2.shape[0] // NW
    nstage = 2
    hcn = cn2 // nstage

    @functools.partial(
        pl.kernel, mesh=_MESH,
        out_type=jax.ShapeDtypeStruct((2, NP, H), F32),
        scratch_types=(
            [pltpu.VMEM_SHARED((NP, H), F32),
             pltpu.VMEM((hcn, ch), jnp.int32),
             pltpu.VMEM((hcn, ch), jnp.int32)]
            + [pltpu.VMEM((ch, H), F32) for _ in range(nb)]
            + [pltpu.SemaphoreType.DMA for _ in range(nb)]
        ),
    )
    def k(hp_hbm, src_hbm, dst_hbm, z_hbm, out_hbm, acc, sidx, didx,
          *bufs_sems):
        rows = bufs_sems[:nb]
        sems = bufs_sems[nb:]
        c = lax.axis_index("c")
        s = lax.axis_index("s")
        w = s * 2 + c
        pltpu.sync_copy(z_hbm.at[pl.ds(0, ch)], rows[0])
        for t in range(stripe // ch):
            pltpu.sync_copy(rows[0], acc.at[pl.ds(s * stripe + t * ch, ch)])
        plsc.subcore_barrier()

        def start(j, p):
            pltpu.async_copy(hp_hbm.at[sidx.at[j]], rows[p], sems[p])

        def drain(j, p):
            pltpu.make_async_copy(hp_hbm.at[sidx.at[j]], rows[p], sems[p]).wait()

        def process(j, p):
            drain(j, p)
            pltpu.sync_copy(rows[p], acc.at[didx.at[j]], add=True)

        def body(jj, carry):
            for p in range(nb):
                j = jj * nb + p
                start(lax.rem(j + nb - 1, hcn), (p + nb - 1) % nb)
                process(j, p)
            return carry

        for half in range(nstage):
            base = w * cn2 + half * hcn
            pltpu.sync_copy(src_hbm.at[pl.ds(base, hcn)], sidx)
            pltpu.sync_copy(dst_hbm.at[pl.ds(base, hcn)], didx)
            for i in range(nb - 1):
                start(i, i)
            lax.fori_loop(0, hcn // nb, body, 0)
            for i in range(nb - 1):
                drain(i, i)
        plsc.subcore_barrier()
        for t in range(stripe // ch):
            sl = pl.ds(s * stripe + t * ch, ch)
            pltpu.sync_copy(acc.at[sl], rows[t % 2])
            pltpu.sync_copy(rows[t % 2], out_hbm.at[c, sl])

    return k(hp, src2, dst2, zrows)


def _sc_degree(dst2, orows, zrows):
    stripe = NP // 16

    @functools.partial(
        pl.kernel, mesh=_MESH,
        out_type=jax.ShapeDtypeStruct((2, NP, H), F32),
        scratch_types=[
            pltpu.VMEM_SHARED((NP, H), F32),
            pltpu.VMEM((CN, CHUNK), jnp.int32),
            pltpu.VMEM((CHUNK, H), F32),
            pltpu.VMEM((CHUNK, H), F32),
            pltpu.SemaphoreType.DMA,
        ],
    )
    def k(dst_hbm, ones_hbm, z_hbm, out_hbm, acc, didx, ones, zero, sem):
        c = lax.axis_index("c")
        s = lax.axis_index("s")
        w = s * 2 + c
        pltpu.sync_copy(dst_hbm.at[pl.ds(w * CN, CN)], didx)
        pltpu.sync_copy(ones_hbm, ones)
        pltpu.sync_copy(z_hbm, zero)
        for t in range(stripe // CHUNK):
            pltpu.sync_copy(zero, acc.at[pl.ds(s * stripe + t * CHUNK, CHUNK)])
        plsc.subcore_barrier()

        def body(j, carry):
            pltpu.sync_copy(ones, acc.at[didx.at[j]], add=True)
            return carry

        lax.fori_loop(0, CN, body, 0)
        plsc.subcore_barrier()
        for t in range(stripe // CHUNK):
            sl = pl.ds(s * stripe + t * CHUNK, CHUNK)
            pltpu.sync_copy(acc.at[sl], ones)
            pltpu.sync_copy(ones, out_hbm.at[c, sl])

    return k(dst2, orows, zrows)


def _deg_reduce_body(p0_ref, p1_ref, o_ref):
    s = p0_ref[0][:, 0:1] + p1_ref[0][:, 0:1]
    o_ref[...] = lax.rsqrt(s + 1.0)


def _dinv_from_parts(parts, *, bn=2048):
    return pl.pallas_call(
        _deg_reduce_body,
        grid=(NP // bn,),
        in_specs=[
            pl.BlockSpec((1, bn, H), lambda i: (0, i, 0)),
            pl.BlockSpec((1, bn, H), lambda i: (1, i, 0)),
        ],
        out_specs=pl.BlockSpec((bn, 1), lambda i: (i, 0)),
        out_shape=jax.ShapeDtypeStruct((NP, 1), F32),
    )(parts, parts)


def _mm_body(a_ref, w_ref, b_ref, o_ref, *, relu, scale_ref=None):
    acc = jnp.dot(a_ref[...], w_ref[...], preferred_element_type=F32)
    acc = acc + b_ref[...]
    if scale_ref is not None:
        acc = acc * scale_ref[...]
    if relu:
        acc = jnp.maximum(acc, 0.0)
    o_ref[...] = acc


def _mm(a, w, b, *, relu=False, scale=None, bm=2000):
    m, k = a.shape
    n = w.shape[1]
    assert m % bm == 0, (m, bm)
    b2 = jnp.reshape(b, (1, n))
    in_specs = [
        pl.BlockSpec((bm, k), lambda i: (i, 0)),
        pl.BlockSpec((k, n), lambda i: (0, 0)),
        pl.BlockSpec((1, n), lambda i: (0, 0)),
    ]
    args = [a, w, b2]
    if scale is not None:
        in_specs.append(pl.BlockSpec((bm, 1), lambda i: (i, 0)))
        args.append(jnp.reshape(scale, (m, 1)))
        body = functools.partial(_mm_body_scaled, relu=relu)
    else:
        body = functools.partial(_mm_body, relu=relu)
    return pl.pallas_call(
        body,
        grid=(m // bm,),
        in_specs=in_specs,
        out_specs=pl.BlockSpec((bm, n), lambda i: (i, 0)),
        out_shape=jax.ShapeDtypeStruct((m, n), F32),
    )(*args)


def _mm_body_scaled(a_ref, w_ref, b_ref, s_ref, o_ref, *, relu):
    acc = jnp.dot(a_ref[...], w_ref[...], preferred_element_type=F32)
    acc = acc + b_ref[...]
    acc = acc * s_ref[...]
    if relu:
        acc = jnp.maximum(acc, 0.0)
    o_ref[...] = acc


def _gcn_combine_body(p0_ref, p1_ref, hp_ref, d_ref, b_ref, o_ref):
    p = p0_ref[0] + p1_ref[0]
    o_ref[...] = jnp.maximum(
        (p + hp_ref[...]) * d_ref[...] + b_ref[...], 0.0)


def _gcn_combine(parts, hp, dinv, b, *, bm=2000):
    m, n = hp.shape
    return pl.pallas_call(
        _gcn_combine_body,
        grid=(m // bm,),
        in_specs=[
            pl.BlockSpec((1, bm, n), lambda i: (0, i, 0)),
            pl.BlockSpec((1, bm, n), lambda i: (1, i, 0)),
            pl.BlockSpec((bm, n), lambda i: (i, 0)),
            pl.BlockSpec((bm, 1), lambda i: (i, 0)),
            pl.BlockSpec((1, n), lambda i: (0, 0)),
        ],
        out_specs=pl.BlockSpec((bm, n), lambda i: (i, 0)),
        out_shape=jax.ShapeDtypeStruct((m, n), F32),
    )(parts, parts, hp, jnp.reshape(dinv, (m, 1)), jnp.reshape(b, (1, n)))


def _gru_body(xe_ref, wih_ref, whh_ref, bih_ref, bhh_ref, o_ref):
    bn = xe_ref.shape[0]
    wih = wih_ref[...]
    whh = whh_ref[...]
    bih = bih_ref[...]
    bhh = bhh_ref[...]
    xe = xe_ref[...]

    h = jnp.zeros((bn, H), F32)
    acc = jnp.zeros((bn, H), F32)
    for t in range(L):
        xt = xe[:, t, :]
        gi = jnp.dot(xt, wih, preferred_element_type=F32) + bih
        gh = jnp.dot(h, whh, preferred_element_type=F32) + bhh
        i_r = gi[:, 0:H]
        i_z = gi[:, H:2 * H]
        i_n = gi[:, 2 * H:3 * H]
        h_r = gh[:, 0:H]
        h_z = gh[:, H:2 * H]
        h_n = gh[:, 2 * H:3 * H]
        r = jax.nn.sigmoid(i_r + h_r)
        z = jax.nn.sigmoid(i_z + h_z)
        ncand = jnp.tanh(i_n + r * h_n)
        h = (1.0 - z) * ncand + z * h
        acc = acc + h
    o_ref[...] = acc


def _gru_sum(xe, gWih, gWhh, gbih, gbhh, *, bn=1000):
    n = xe.shape[0]
    wih = jnp.pad(gWih.T, ((0, 128 - TD), (0, 0)))
    whh = gWhh.T
    return pl.pallas_call(
        _gru_body,
        grid=(n // bn,),
        in_specs=[
            pl.BlockSpec((bn, L, 128), lambda i: (i, 0, 0)),
            pl.BlockSpec((128, 3 * H), lambda i: (0, 0)),
            pl.BlockSpec((H, 3 * H), lambda i: (0, 0)),
            pl.BlockSpec((1, 3 * H), lambda i: (0, 0)),
            pl.BlockSpec((1, 3 * H), lambda i: (0, 0)),
        ],
        out_specs=pl.BlockSpec((bn, H), lambda i: (i, 0)),
        out_shape=jax.ShapeDtypeStruct((n, H), F32),
    )(xe, wih, whh, jnp.reshape(gbih, (1, 3 * H)), jnp.reshape(gbhh, (1, 3 * H)))


def _im2col_s2(x):
    hi, wi, c = x.shape
    ho, wo = hi // 2, wi // 2
    xp = jnp.pad(x, ((0, 1), (0, 1), (0, 0)))
    slabs = []
    for dy in range(3):
        for dx in range(3):
            slabs.append(xp[dy:dy + hi:2, dx:dx + wi:2, :])
    a = jnp.stack(slabs, axis=2)
    return jnp.reshape(a, (ho * wo, 9 * c))


def _conv_gemm(x, k, cb, *, bm):
    hi, wi, cin = x.shape
    cout = k.shape[0]
    a = _im2col_s2(x)
    wmat = jnp.reshape(jnp.transpose(k, (2, 3, 1, 0)), (9 * cin, cout))
    out = _mm(a, wmat, cb, relu=True, bm=bm)
    return jnp.reshape(out, (hi // 2, wi // 2, cout))


def _edge_body(us_ref, ud_ref, bcat_ref, r1_ref, r1b_ref, r2_ref, r2b_ref,
               o_ref):
    e = jnp.maximum(us_ref[...] + ud_ref[...] + bcat_ref[...], 0.0)
    h = jnp.maximum(
        jnp.dot(e, r1_ref[...], preferred_element_type=F32) + r1b_ref[...], 0.0)
    logits = jnp.dot(h, r2_ref[...], preferred_element_type=F32) + r2b_ref[...]
    m = jnp.max(logits, axis=1, keepdims=True)
    lse = jnp.log(jnp.sum(jnp.exp(logits - m), axis=1, keepdims=True)) + m
    o_ref[...] = logits - lse


def _edge_head(us, ud, bcat, r1W, r1b, r2W, r2b, *, be=3200):
    e = us.shape[0]
    return pl.pallas_call(
        _edge_body,
        grid=(e // be,),
        in_specs=[
            pl.BlockSpec((be, 3 * H), lambda i: (i, 0)),
            pl.BlockSpec((be, 3 * H), lambda i: (i, 0)),
            pl.BlockSpec((1, 3 * H), lambda i: (0, 0)),
            pl.BlockSpec((3 * H, H), lambda i: (0, 0)),
            pl.BlockSpec((1, H), lambda i: (0, 0)),
            pl.BlockSpec((H, NC), lambda i: (0, 0)),
            pl.BlockSpec((1, NC), lambda i: (0, 0)),
        ],
        out_specs=pl.BlockSpec((be, NC), lambda i: (i, 0)),
        out_shape=jax.ShapeDtypeStruct((e, NC), F32),
    )(us, ud, jnp.reshape(bcat, (1, 3 * H)), r1W, jnp.reshape(r1b, (1, H)),
      r2W, jnp.reshape(r2b, (1, NC)))


def _gcn_layer(x, src2, dst2, dinv, zrows, W, b):
    hp = _mm(x, W, jnp.zeros((W.shape[1],), F32), scale=dinv)
    parts = _sc_propagate(hp, src2, dst2, zrows)
    return _gcn_combine(parts, hp, dinv, b)


def kernel(x, edge_index, xtext, img, nodenum, pos, cell_wh, W1, b1, W2, b2,
           Wt1, bt1, Wt2, bt2, emb, gWih, gWhh, gbih, gbhh, K1, cb1, K2, cb2,
           K3, cb3, lpW, lpb, ltW, ltb, liW, lib, r1W, r1b, r2W, r2b):
    src = edge_index[0]
    dst = edge_index[1]
    pad_e = EP - E
    i32 = jnp.int32
    srcp = jnp.concatenate([src, jnp.zeros((pad_e,), i32)])
    dstp_sc = jnp.concatenate([dst, jnp.full((pad_e,), NP - 1, i32)])
    dstp_u = jnp.concatenate([dst, jnp.zeros((pad_e,), i32)])
    src2 = jnp.reshape(srcp, (NW * CN, CHUNK))
    dst2sc = jnp.reshape(dstp_sc, (NW * CN, CHUNK))
    dst2u = jnp.reshape(dstp_u, (NW * CN, CHUNK))
    zrows = jnp.zeros((CHUNK, H), F32)

    orows = jnp.ones((CHUNK, H), F32)
    degp = _sc_degree(dst2sc, orows, zrows)
    dinv = _dinv_from_parts(degp)[:N]

    pf = _gcn_layer(x, src2, dst2sc, dinv, zrows, W1, b1)
    pf = _gcn_layer(pf, src2, dst2sc, dinv, zrows, W2, b2)

    nlp = NW * 40 * CHUNK
    tok = jnp.reshape(xtext, (-1,))
    tok2 = jnp.reshape(
        jnp.concatenate([tok, jnp.zeros((nlp - N * L,), i32)]), (NW * 40, CHUNK))
    emb128 = jnp.pad(emb, ((0, 0), (0, 128 - TD)))
    xef = _sc_gather(emb128, tok2)
    xe = jnp.reshape(xef[:N * L], (N, L, 128))
    tf = _gru_sum(xe, gWih, gWhh, gbih, gbhh)
    tf = _gcn_layer(tf, src2, dst2sc, dinv, zrows, Wt1, bt1)
    tf = _gcn_layer(tf, src2, dst2sc, dinv, zrows, Wt2, bt2)

    xim = jnp.transpose(img[0], (1, 2, 0))
    fm = _conv_gemm(xim, K1, cb1, bm=4096)
    fm = _conv_gemm(fm, K2, cb2, bm=4096)
    fm = _conv_gemm(fm, K3, cb3, bm=4096)
    fmflat = jnp.reshape(fm, (64 * 64, H))

    hf = wf = 64
    idx_parts = []
    for s in range(NS):
        t = (s + 0.5) / NS
        q = pos - cell_wh / 2.0 + t * cell_wh
        ix = jnp.clip((jnp.clip(q[:, 0], 0.0, 1.0) * wf).astype(i32), 0, wf - 1)
        iy = jnp.clip((jnp.clip(q[:, 1], 0.0, 1.0) * hf).astype(i32), 0, hf - 1)
        idx_parts.append(
            jnp.concatenate([iy * wf + ix, jnp.zeros((NP - N,), i32)]))
    idx_parts.append(jnp.zeros((NW * 8 * CHUNK - 2 * NP,), i32))
    idx2 = jnp.reshape(jnp.concatenate(idx_parts), (NW * 8, CHUNK))
    g = _sc_gather(fmflat, idx2)
    g0 = g[:N]
    g1 = g[NP:NP + N]

    feat = jnp.concatenate([pf, tf, g0, g1], axis=1)
    z = jnp.zeros((H, H), F32)
    ws = jnp.concatenate([
        jnp.concatenate([lpW[:H], z, z], axis=1),
        jnp.concatenate([z, ltW[:H], z], axis=1),
        jnp.concatenate([z, z, liW[0:H]], axis=1),
        jnp.concatenate([z, z, liW[H:2 * H]], axis=1),
    ], axis=0)
    wd = jnp.concatenate([
        jnp.concatenate([lpW[H:], z, z], axis=1),
        jnp.concatenate([z, ltW[H:], z], axis=1),
        jnp.concatenate([z, z, liW[2 * H:3 * H]], axis=1),
        jnp.concatenate([z, z, liW[3 * H:4 * H]], axis=1),
    ], axis=0)
    zb = jnp.zeros((3 * H,), F32)
    S = _mm(feat, ws, zb)
    D = _mm(feat, wd, zb)

    us = _sc_gather(S, src2, nb=2)
    ud = _sc_gather(D, dst2u, nb=2)
    bcat = jnp.concatenate([lpb, ltb, lib])
    out = _edge_head(us, ud, bcat, r1W, r1b, r2W, r2b, be=4096)
    return out[:E]
